# baseline (speedup 1.0000x reference)
import functools

import jax
import jax.numpy as jnp
from jax import lax
from jax.experimental import pallas as pl
from jax.experimental.pallas import tpu as pltpu
from jax.experimental.pallas import tpu_sc as plsc

NC = 2
NS = 16
NW = NC * NS
L = 16
BATCH = 16384
K = 16
BPW = BATCH // NW
CHUNK = 128
NCHUNK = BPW // CHUNK


def _mf_body(uidx_hbm, vidx_hbm, w_hbm, h_hbm, out_hbm,
             uidx_v, vidx_v, urows_v, vrows_v, out_v, sem):
    wid = lax.axis_index("s") * NC + lax.axis_index("c")
    base = wid * BPW

    pltpu.sync_copy(uidx_hbm.at[wid], uidx_v)
    pltpu.sync_copy(vidx_hbm.at[wid], vidx_v)

    copies = []
    for j in range(NCHUNK):
        cu = pltpu.async_copy(w_hbm.at[uidx_v.at[j]],
                              urows_v.at[pl.ds(j * CHUNK, CHUNK)], sem)
        cv = pltpu.async_copy(h_hbm.at[vidx_v.at[j]],
                              vrows_v.at[pl.ds(j * CHUNK, CHUNK)], sem)
        copies.append((cu, cv))

    for j in range(NCHUNK):
        cu, cv = copies[j]
        cu.wait()
        cv.wait()

        def blk_body(b, carry, j=j):
            start = j * CHUNK + b * L
            row = start + lax.broadcasted_iota(jnp.int32, (L,), 0)
            acc = jnp.zeros((L,), jnp.float32)
            for k in range(K):
                col = jnp.full((L,), k, jnp.int32)
                u = plsc.load_gather(urows_v, [row, col])
                v = plsc.load_gather(vrows_v, [row, col])
                acc = acc + u * v
            out_v[pl.ds(start, L)] = acc
            return carry

        lax.fori_loop(0, CHUNK // L, blk_body, 0)

    pltpu.sync_copy(out_v, out_hbm.at[pl.ds(base, BPW)])


@jax.jit
def kernel(x, W, H):
    u_idx = x[:, 0].astype(jnp.int32).reshape(NW, NCHUNK, CHUNK)
    v_idx = x[:, 1].astype(jnp.int32).reshape(NW, NCHUNK, CHUNK)
    mf = functools.partial(
        pl.kernel,
        mesh=plsc.VectorSubcoreMesh(core_axis_name="c", subcore_axis_name="s"),
        out_type=jax.ShapeDtypeStruct((BATCH,), jnp.float32),
        compiler_params=pltpu.CompilerParams(
            needs_layout_passes=False, use_tc_tiling_on_sc=False),
        scratch_types=[
            pltpu.VMEM((NCHUNK, CHUNK), jnp.int32),
            pltpu.VMEM((NCHUNK, CHUNK), jnp.int32),
            pltpu.VMEM((BPW, K), jnp.float32),
            pltpu.VMEM((BPW, K), jnp.float32),
            pltpu.VMEM((BPW,), jnp.float32),
            pltpu.SemaphoreType.DMA,
        ],
    )(_mf_body)
    return mf(u_idx, v_idx, W, H)
